# Initial kernel scaffold; baseline (speedup 1.0000x reference)
import jax
import jax.numpy as jnp
from jax.experimental import pallas as pl


def kernel(x, pos, edge_attr, params, edge_index, batch):
    raise NotImplementedError("write your pallas kernel here")


# speedup vs baseline: 2.2693x; 2.2693x over previous
import functools

import jax
import jax.numpy as jnp
from jax import lax
from jax.experimental import pallas as pl
from jax.experimental.pallas import tpu as pltpu
from jax.experimental.pallas import tpu_sc as plsc

N = 10000
E = 320000
G = 32
DP = 128
ONES_COL = 56

_HI = lax.Precision.HIGHEST


def _dot(a, b):
    return jnp.dot(a, b, preferred_element_type=jnp.float32, precision=_HI)


def _encode_body(x_ref, pos_ref, w1t_ref, b1_ref, w2t_ref, b2_ref, out_ref):
    pos = pos_ref[...]
    x = x_ref[...]
    r = jnp.sqrt(jnp.sum(pos * pos, axis=1, keepdims=True))
    px, py = pos[:, 0:1], pos[:, 1:2]
    inv = jnp.concatenate([r, px * px + py * py, px * py], axis=1)
    g1 = jnp.maximum(_dot(inv, w1t_ref[...]) + b1_ref[...], 0.0)
    geo = _dot(g1, w2t_ref[...]) + b2_ref[...]
    ones = jnp.ones((N, 1), jnp.float32)
    zeros = jnp.zeros((N, DP - ONES_COL - 1), jnp.float32)
    out_ref[...] = jnp.concatenate([x, geo, ones, zeros], axis=1)


def _encode(x, pos, p):
    return pl.pallas_call(
        _encode_body,
        out_shape=jax.ShapeDtypeStruct((N, DP), jnp.float32),
    )(x, pos, p['enc_w1'].T, p['enc_b1'][None], p['enc_w2'].T, p['enc_b2'][None])


EB = 4000


def _edge_mlp_body(set_ones, ea_ref, w1t_ref, b1_ref, g_ref, be_ref,
                   w2t_ref, b2_ref, out_ref):
    h = _dot(ea_ref[...], w1t_ref[...]) + b1_ref[...]
    mu = jnp.mean(h, axis=1, keepdims=True)
    var = jnp.mean((h - mu) ** 2, axis=1, keepdims=True)
    h = (h - mu) / jnp.sqrt(var + 1e-5) * g_ref[...] + be_ref[...]
    h = jnp.maximum(h, 0.0)
    ew = _dot(h, w2t_ref[...]) + b2_ref[...]
    if set_ones:
        col = lax.broadcasted_iota(jnp.int32, ew.shape, 1)
        ew = jnp.where(col == ONES_COL, 1.0, ew)
    out_ref[...] = ew


def _edge_mlp(ea, cp, set_ones):
    in_d = cp['em_w2'].shape[0]
    w1t = jnp.pad(cp['em_w1'].T, ((0, 3), (0, 0)))
    w2t = jnp.pad(cp['em_w2'].T, ((0, 0), (0, DP - in_d)))
    b2 = jnp.pad(cp['em_b2'], (0, DP - in_d))[None]
    fixed = lambda i: (0, 0)
    return pl.pallas_call(
        functools.partial(_edge_mlp_body, set_ones),
        grid=(E // EB,),
        in_specs=[
            pl.BlockSpec((EB, 8), lambda i: (i, 0)),
            pl.BlockSpec((8, 16), fixed),
            pl.BlockSpec((1, 16), fixed),
            pl.BlockSpec((1, 16), fixed),
            pl.BlockSpec((1, 16), fixed),
            pl.BlockSpec((16, DP), fixed),
            pl.BlockSpec((1, DP), fixed),
        ],
        out_specs=pl.BlockSpec((EB, DP), lambda i: (i, 0)),
        out_shape=jax.ShapeDtypeStruct((E, DP), jnp.float32),
    )(ea, w1t, cp['em_b1'][None], cp['ln_g'][None], cp['ln_b'][None], w2t, b2)


CH = 80
NW = 32
EPT = E // NW
NCH = EPT // CH
ZR = 125


def _sc_scatter(h, ew, src, dst):
    mesh = plsc.VectorSubcoreMesh(core_axis_name="c", subcore_axis_name="s")

    @functools.partial(
        pl.kernel,
        out_type=jax.ShapeDtypeStruct((2, N, DP), jnp.float32),
        mesh=mesh,
        scratch_types=[
            pltpu.VMEM((CH,), jnp.int32),
            pltpu.VMEM((CH,), jnp.int32),
            pltpu.VMEM((CH, DP), jnp.float32),
            pltpu.VMEM((CH, DP), jnp.float32),
            pltpu.VMEM((ZR, DP), jnp.float32),
            pltpu.VMEM_SHARED((N, DP), jnp.float32),
            pltpu.SemaphoreType.DMA,
        ],
    )
    def k(h_hbm, ew_hbm, src_hbm, dst_hbm, t_hbm,
          src_v, dst_v, ew_v, rows_v, z_v, t_sp, sem):
        c = lax.axis_index("c")
        s = lax.axis_index("s")

        zero = jnp.zeros((16,), jnp.float32)

        def zrow(r, carry):
            for j in range(DP // 16):
                z_v[r, pl.ds(j * 16, 16)] = zero
            return carry

        lax.fori_loop(0, ZR, zrow, 0)
        for q in range(5):
            pltpu.sync_copy(z_v, t_sp.at[pl.ds(s * 625 + q * ZR, ZR)])
        plsc.subcore_barrier()

        base = c * (E // 2) + s * EPT

        def chunk(i, carry):
            off = base + i * CH
            pltpu.sync_copy(src_hbm.at[pl.ds(off, CH)], src_v)
            pltpu.sync_copy(dst_hbm.at[pl.ds(off, CH)], dst_v)
            pltpu.sync_copy(ew_hbm.at[pl.ds(off, CH)], ew_v)
            pltpu.async_copy(h_hbm.at[src_v], rows_v, sem).wait()

            def prow(r, cr):
                for j in range(DP // 16):
                    sl = pl.ds(j * 16, 16)
                    rows_v[r, sl] = rows_v[r, sl] * ew_v[r, sl]
                return cr

            lax.fori_loop(0, CH, prow, 0)
            pltpu.sync_copy(rows_v, t_sp.at[dst_v], add=True)
            return carry

        lax.fori_loop(0, NCH, chunk, 0)
        plsc.subcore_barrier()

        @pl.when(s == 0)
        def _():
            pltpu.sync_copy(t_sp, t_hbm.at[c])

    return k(h, ew, src, dst)


NB = 2000


def _aggr_node(t0, t1, cnt, h, epwt, epb, nmwt, nmb):
    t = t0 + t1
    tm = t / jnp.maximum(cnt, 1.0)
    aggr = _dot(tm, epwt) + jnp.where(cnt > 0.0, 1.0, 0.0) * epb
    return _dot(h, nmwt) + nmb + aggr


def _c1y_body(t0_ref, t1_ref, h_ref, epwt_ref, epb_ref, nmwt_ref, nmb_ref,
              y_ref, cnt_out_ref):
    t0 = t0_ref[...]
    t1 = t1_ref[...]
    cnt = t0[:, ONES_COL:ONES_COL + 1] + t1[:, ONES_COL:ONES_COL + 1]
    y_ref[...] = _aggr_node(t0, t1, cnt, h_ref[...], epwt_ref[...],
                            epb_ref[...], nmwt_ref[...], nmb_ref[...])
    cnt_out_ref[...] = jnp.broadcast_to(cnt, (NB, 8))


def _c2y_body(t0_ref, t1_ref, h_ref, cnt_ref, epwt_ref, epb_ref, nmwt_ref,
              nmb_ref, y_ref):
    cnt = cnt_ref[:, 0:1]
    y_ref[...] = _aggr_node(t0_ref[...], t1_ref[...], cnt, h_ref[...],
                            epwt_ref[...], epb_ref[...], nmwt_ref[...],
                            nmb_ref[...])


def _bn_body(y_ref, g_ref, b_ref, out_ref):
    y = y_ref[...]
    m = jnp.mean(y, axis=0, keepdims=True)
    v = jnp.mean((y - m) ** 2, axis=0, keepdims=True)
    out_ref[...] = jnp.maximum(
        (y - m) / jnp.sqrt(v + 1e-5) * g_ref[...] + b_ref[...], 0.0)


def _c3pool_body(t0_ref, t1_ref, h_ref, cnt_ref, batch_ref, epwt_ref,
                 epb_ref, nmwt_ref, nmb_ref, out_ref, cnt_acc):
    i = pl.program_id(0)
    cnt = cnt_ref[:, 0:1]
    y = _aggr_node(t0_ref[...], t1_ref[...], cnt, h_ref[...], epwt_ref[...],
                   epb_ref[...], nmwt_ref[...], nmb_ref[...])
    bid = batch_ref[0]
    gi = lax.broadcasted_iota(jnp.int32, (G, NB), 0)
    mask = (gi == bid).astype(jnp.float32)

    @pl.when(i == 0)
    def _():
        out_ref[...] = jnp.zeros_like(out_ref)
        cnt_acc[...] = jnp.zeros_like(cnt_acc)

    out_ref[...] += _dot(mask, y)
    cnt_acc[...] += jnp.broadcast_to(
        jnp.sum(mask, axis=1, keepdims=True), (G, 8))

    @pl.when(i == pl.num_programs(0) - 1)
    def _():
        out_ref[...] = out_ref[...] / jnp.maximum(cnt_acc[:, 0:1], 1.0)


def _pad_nodew(w, out_d):
    return jnp.pad(w.T, ((0, DP - w.shape[1]), (0, 0)))


def _fixed(i):
    return (0, 0)


def _rowblk(i):
    return (i, 0)


def _node_layer(t, h, cnt8, cp, first):
    out_d = cp['ep_w'].shape[0]
    epwt = _pad_nodew(cp['ep_w'], out_d)
    nmwt = _pad_nodew(cp['nm_w'], out_d)
    wspecs = [
        pl.BlockSpec((DP, out_d), _fixed),
        pl.BlockSpec((1, out_d), _fixed),
        pl.BlockSpec((DP, out_d), _fixed),
        pl.BlockSpec((1, out_d), _fixed),
    ]
    tspec = pl.BlockSpec((NB, DP), _rowblk)
    if first:
        return pl.pallas_call(
            _c1y_body,
            grid=(N // NB,),
            in_specs=[tspec, tspec, tspec] + wspecs,
            out_specs=(pl.BlockSpec((NB, out_d), _rowblk),
                       pl.BlockSpec((NB, 8), _rowblk)),
            out_shape=(jax.ShapeDtypeStruct((N, out_d), jnp.float32),
                       jax.ShapeDtypeStruct((N, 8), jnp.float32)),
        )(t[0], t[1], h, epwt, cp['ep_b'][None], nmwt, cp['nm_b'][None])
    return pl.pallas_call(
        _c2y_body,
        grid=(N // NB,),
        in_specs=[tspec, tspec, tspec, pl.BlockSpec((NB, 8), _rowblk)]
        + wspecs,
        out_specs=pl.BlockSpec((NB, out_d), _rowblk),
        out_shape=jax.ShapeDtypeStruct((N, out_d), jnp.float32),
    )(t[0], t[1], h, cnt8, epwt, cp['ep_b'][None], nmwt, cp['nm_b'][None])


def _bn_relu(y, g, b):
    return pl.pallas_call(
        _bn_body,
        out_shape=jax.ShapeDtypeStruct(y.shape, jnp.float32),
    )(y, g[None], b[None])


def kernel(x, pos, edge_attr, params, edge_index, batch):
    p = params
    src = edge_index[0]
    dst = edge_index[1]
    ea = jnp.pad(edge_attr, ((0, 0), (0, 3)))
    batch_row = batch.astype(jnp.int32).reshape(N // NB, 1, NB)

    h0 = _encode(x, pos, p)

    c1 = p['conv1']
    ew1 = _edge_mlp(ea, c1, True)
    t1 = _sc_scatter(h0, ew1, src, dst)
    y1, cnt8 = _node_layer(t1, h0, None, c1, True)
    h1 = _bn_relu(y1, p['bn1_g'], p['bn1_b'])

    c2 = p['conv2']
    ew2 = _edge_mlp(ea, c2, False)
    t2 = _sc_scatter(h1, ew2, src, dst)
    y2 = _node_layer(t2, h1, cnt8, c2, False)
    h2 = _bn_relu(y2, p['bn2_g'], p['bn2_b'])

    c3 = p['conv3']
    ew3 = _edge_mlp(ea, c3, False)
    t3 = _sc_scatter(h2, ew3, src, dst)
    epwt3 = _pad_nodew(c3['ep_w'], 256)
    nmwt3 = _pad_nodew(c3['nm_w'], 256)
    tspec = pl.BlockSpec((NB, DP), _rowblk)
    gf = pl.pallas_call(
        _c3pool_body,
        grid=(N // NB,),
        in_specs=[tspec, tspec, tspec,
                  pl.BlockSpec((NB, 8), _rowblk),
                  pl.BlockSpec((1, 1, NB), lambda i: (i, 0, 0)),
                  pl.BlockSpec((DP, 256), _fixed),
                  pl.BlockSpec((1, 256), _fixed),
                  pl.BlockSpec((DP, 256), _fixed),
                  pl.BlockSpec((1, 256), _fixed)],
        out_specs=pl.BlockSpec((G, 256), _fixed),
        out_shape=jax.ShapeDtypeStruct((G, 256), jnp.float32),
        scratch_shapes=[pltpu.VMEM((G, 8), jnp.float32)],
    )(t3[0], t3[1], h2, cnt8, batch_row, epwt3, c3['ep_b'][None],
      nmwt3, c3['nm_b'][None])

    return gf


# speedup vs baseline: 2.5755x; 1.1349x over previous
import functools

import jax
import jax.numpy as jnp
from jax import lax
from jax.experimental import pallas as pl
from jax.experimental.pallas import tpu as pltpu
from jax.experimental.pallas import tpu_sc as plsc

N = 10000
E = 320000
G = 32
DP = 128
ONES_COL = 56

_HI = lax.Precision.HIGHEST


def _dot(a, b):
    return jnp.dot(a, b, preferred_element_type=jnp.float32, precision=_HI)


def _encode_body(x_ref, pos_ref, w1t_ref, b1_ref, w2t_ref, b2_ref, out_ref):
    pos = pos_ref[...]
    x = x_ref[...]
    r = jnp.sqrt(jnp.sum(pos * pos, axis=1, keepdims=True))
    px, py = pos[:, 0:1], pos[:, 1:2]
    inv = jnp.concatenate([r, px * px + py * py, px * py], axis=1)
    g1 = jnp.maximum(_dot(inv, w1t_ref[...]) + b1_ref[...], 0.0)
    geo = _dot(g1, w2t_ref[...]) + b2_ref[...]
    ones = jnp.ones((N, 1), jnp.float32)
    zeros = jnp.zeros((N, DP - ONES_COL - 1), jnp.float32)
    out_ref[...] = jnp.concatenate([x, geo, ones, zeros], axis=1)


def _encode(x, pos, p):
    return pl.pallas_call(
        _encode_body,
        out_shape=jax.ShapeDtypeStruct((N, DP), jnp.float32),
    )(x, pos, p['enc_w1'].T, p['enc_b1'][None], p['enc_w2'].T, p['enc_b2'][None])


EB = 4000


def _edge_mlp_body(set_ones, ea_ref, w1t_ref, b1_ref, g_ref, be_ref,
                   w2t_ref, b2_ref, out_ref):
    h = _dot(ea_ref[...], w1t_ref[...]) + b1_ref[...]
    mu = jnp.mean(h, axis=1, keepdims=True)
    var = jnp.mean((h - mu) ** 2, axis=1, keepdims=True)
    h = (h - mu) / jnp.sqrt(var + 1e-5) * g_ref[...] + be_ref[...]
    h = jnp.maximum(h, 0.0)
    ew = _dot(h, w2t_ref[...]) + b2_ref[...]
    if set_ones:
        col = lax.broadcasted_iota(jnp.int32, ew.shape, 1)
        ew = jnp.where(col == ONES_COL, 1.0, ew)
    out_ref[...] = ew


def _edge_mlp(ea, cp, set_ones):
    in_d = cp['em_w2'].shape[0]
    w1t = jnp.pad(cp['em_w1'].T, ((0, 3), (0, 0)))
    w2t = jnp.pad(cp['em_w2'].T, ((0, 0), (0, DP - in_d)))
    b2 = jnp.pad(cp['em_b2'], (0, DP - in_d))[None]
    fixed = lambda i: (0, 0)
    return pl.pallas_call(
        functools.partial(_edge_mlp_body, set_ones),
        grid=(E // EB,),
        in_specs=[
            pl.BlockSpec((EB, 8), lambda i: (i, 0)),
            pl.BlockSpec((8, 16), fixed),
            pl.BlockSpec((1, 16), fixed),
            pl.BlockSpec((1, 16), fixed),
            pl.BlockSpec((1, 16), fixed),
            pl.BlockSpec((16, DP), fixed),
            pl.BlockSpec((1, DP), fixed),
        ],
        out_specs=pl.BlockSpec((EB, DP), lambda i: (i, 0)),
        out_shape=jax.ShapeDtypeStruct((E, DP), jnp.float32),
    )(ea, w1t, cp['em_b1'][None], cp['ln_g'][None], cp['ln_b'][None], w2t, b2)


CH = 40
NW = 32
EPT = E // NW
NCH = EPT // CH
ZR = 25


def _sc_scatter(h, ew, src, dst):
    mesh = plsc.VectorSubcoreMesh(core_axis_name="c", subcore_axis_name="s")

    @functools.partial(
        pl.kernel,
        out_type=jax.ShapeDtypeStruct((2, N, DP), jnp.float32),
        mesh=mesh,
        scratch_types=[
            pltpu.VMEM((EPT,), jnp.int32),
            pltpu.VMEM((CH,), jnp.int32),
            pltpu.VMEM((CH,), jnp.int32),
            pltpu.VMEM((CH, DP), jnp.float32),
            pltpu.VMEM((CH, DP), jnp.float32),
            pltpu.VMEM((CH, DP), jnp.float32),
            pltpu.VMEM((CH, DP), jnp.float32),
            pltpu.VMEM((ZR, DP), jnp.float32),
            pltpu.VMEM_SHARED((N, DP), jnp.float32),
            pltpu.SemaphoreType.DMA((2,)),
            pltpu.SemaphoreType.DMA((2,)),
            pltpu.SemaphoreType.DMA((2,)),
        ],
    )
    def k(h_hbm, ew_hbm, src_hbm, dst_hbm, t_hbm,
          src_all, dst_v0, dst_v1, ew_v0, ew_v1, rows_v0, rows_v1, z_v,
          t_sp, sem_d, sem_e, sem_g):
        c = lax.axis_index("c")
        s = lax.axis_index("s")
        dstb = (dst_v0, dst_v1)
        ewb = (ew_v0, ew_v1)
        rowb = (rows_v0, rows_v1)

        zero = jnp.zeros((16,), jnp.float32)

        def zrow(r, carry):
            for j in range(DP // 16):
                z_v[r, pl.ds(j * 16, 16)] = zero
            return carry

        lax.fori_loop(0, ZR, zrow, 0)

        def zcopy(q, carry):
            pltpu.sync_copy(z_v, t_sp.at[pl.ds(s * 625 + q * ZR, ZR)])
            return carry

        lax.fori_loop(0, 625 // ZR, zcopy, 0)

        base = c * (E // 2) + s * EPT
        pltpu.sync_copy(src_hbm.at[pl.ds(base, EPT)], src_all)
        plsc.subcore_barrier()

        def issue(i, b):
            off = base + i * CH
            pltpu.async_copy(dst_hbm.at[pl.ds(off, CH)], dstb[b], sem_d.at[b])
            pltpu.async_copy(ew_hbm.at[pl.ds(off, CH)], ewb[b], sem_e.at[b])
            pltpu.async_copy(h_hbm.at[src_all.at[pl.ds(i * CH, CH)]],
                             rowb[b], sem_g.at[b])

        def process(b):
            pltpu.make_async_copy(ew_hbm.at[pl.ds(0, CH)], ewb[b],
                                  sem_e.at[b]).wait()
            pltpu.make_async_copy(ew_hbm.at[pl.ds(0, CH)], rowb[b],
                                  sem_g.at[b]).wait()

            def prow(r, cr):
                for j in range(DP // 16):
                    sl = pl.ds(j * 16, 16)
                    rowb[b][r, sl] = rowb[b][r, sl] * ewb[b][r, sl]
                return cr

            lax.fori_loop(0, CH, prow, 0)
            pltpu.make_async_copy(dst_hbm.at[pl.ds(0, CH)], dstb[b],
                                  sem_d.at[b]).wait()
            pltpu.sync_copy(rowb[b], t_sp.at[dstb[b]], add=True)

        issue(0, 0)

        def pair(g, carry):
            issue(2 * g + 1, 1)
            process(0)

            @pl.when(2 * g + 2 < NCH)
            def _():
                issue(2 * g + 2, 0)

            process(1)
            return carry

        lax.fori_loop(0, NCH // 2, pair, 0)
        plsc.subcore_barrier()

        @pl.when(s == 0)
        def _():
            pltpu.sync_copy(t_sp, t_hbm.at[c])

    return k(h, ew, src, dst)


NB = 2000


def _aggr_node(t0, t1, cnt, h, epwt, epb, nmwt, nmb):
    t = t0 + t1
    tm = t / jnp.maximum(cnt, 1.0)
    aggr = _dot(tm, epwt) + jnp.where(cnt > 0.0, 1.0, 0.0) * epb
    return _dot(h, nmwt) + nmb + aggr


def _c1y_body(t0_ref, t1_ref, h_ref, epwt_ref, epb_ref, nmwt_ref, nmb_ref,
              y_ref, cnt_out_ref):
    t0 = t0_ref[...]
    t1 = t1_ref[...]
    cnt = t0[:, ONES_COL:ONES_COL + 1] + t1[:, ONES_COL:ONES_COL + 1]
    y_ref[...] = _aggr_node(t0, t1, cnt, h_ref[...], epwt_ref[...],
                            epb_ref[...], nmwt_ref[...], nmb_ref[...])
    cnt_out_ref[...] = jnp.broadcast_to(cnt, (NB, 8))


def _c2y_body(t0_ref, t1_ref, h_ref, cnt_ref, epwt_ref, epb_ref, nmwt_ref,
              nmb_ref, y_ref):
    cnt = cnt_ref[:, 0:1]
    y_ref[...] = _aggr_node(t0_ref[...], t1_ref[...], cnt, h_ref[...],
                            epwt_ref[...], epb_ref[...], nmwt_ref[...],
                            nmb_ref[...])


def _bn_body(y_ref, g_ref, b_ref, out_ref):
    y = y_ref[...]
    m = jnp.mean(y, axis=0, keepdims=True)
    v = jnp.mean((y - m) ** 2, axis=0, keepdims=True)
    out_ref[...] = jnp.maximum(
        (y - m) / jnp.sqrt(v + 1e-5) * g_ref[...] + b_ref[...], 0.0)


def _c3pool_body(t0_ref, t1_ref, h_ref, cnt_ref, batch_ref, epwt_ref,
                 epb_ref, nmwt_ref, nmb_ref, out_ref, cnt_acc):
    i = pl.program_id(0)
    cnt = cnt_ref[:, 0:1]
    y = _aggr_node(t0_ref[...], t1_ref[...], cnt, h_ref[...], epwt_ref[...],
                   epb_ref[...], nmwt_ref[...], nmb_ref[...])
    bid = batch_ref[0]
    gi = lax.broadcasted_iota(jnp.int32, (G, NB), 0)
    mask = (gi == bid).astype(jnp.float32)

    @pl.when(i == 0)
    def _():
        out_ref[...] = jnp.zeros_like(out_ref)
        cnt_acc[...] = jnp.zeros_like(cnt_acc)

    out_ref[...] += _dot(mask, y)
    cnt_acc[...] += jnp.broadcast_to(
        jnp.sum(mask, axis=1, keepdims=True), (G, 8))

    @pl.when(i == pl.num_programs(0) - 1)
    def _():
        out_ref[...] = out_ref[...] / jnp.maximum(cnt_acc[:, 0:1], 1.0)


def _pad_nodew(w, out_d):
    return jnp.pad(w.T, ((0, DP - w.shape[1]), (0, 0)))


def _fixed(i):
    return (0, 0)


def _rowblk(i):
    return (i, 0)


def _node_layer(t, h, cnt8, cp, first):
    out_d = cp['ep_w'].shape[0]
    epwt = _pad_nodew(cp['ep_w'], out_d)
    nmwt = _pad_nodew(cp['nm_w'], out_d)
    wspecs = [
        pl.BlockSpec((DP, out_d), _fixed),
        pl.BlockSpec((1, out_d), _fixed),
        pl.BlockSpec((DP, out_d), _fixed),
        pl.BlockSpec((1, out_d), _fixed),
    ]
    tspec = pl.BlockSpec((NB, DP), _rowblk)
    if first:
        return pl.pallas_call(
            _c1y_body,
            grid=(N // NB,),
            in_specs=[tspec, tspec, tspec] + wspecs,
            out_specs=(pl.BlockSpec((NB, out_d), _rowblk),
                       pl.BlockSpec((NB, 8), _rowblk)),
            out_shape=(jax.ShapeDtypeStruct((N, out_d), jnp.float32),
                       jax.ShapeDtypeStruct((N, 8), jnp.float32)),
        )(t[0], t[1], h, epwt, cp['ep_b'][None], nmwt, cp['nm_b'][None])
    return pl.pallas_call(
        _c2y_body,
        grid=(N // NB,),
        in_specs=[tspec, tspec, tspec, pl.BlockSpec((NB, 8), _rowblk)]
        + wspecs,
        out_specs=pl.BlockSpec((NB, out_d), _rowblk),
        out_shape=jax.ShapeDtypeStruct((N, out_d), jnp.float32),
    )(t[0], t[1], h, cnt8, epwt, cp['ep_b'][None], nmwt, cp['nm_b'][None])


def _bn_relu(y, g, b):
    return pl.pallas_call(
        _bn_body,
        out_shape=jax.ShapeDtypeStruct(y.shape, jnp.float32),
    )(y, g[None], b[None])


def kernel(x, pos, edge_attr, params, edge_index, batch):
    p = params
    src = edge_index[0]
    dst = edge_index[1]
    ea = jnp.pad(edge_attr, ((0, 0), (0, 3)))
    batch_row = batch.astype(jnp.int32).reshape(N // NB, 1, NB)

    h0 = _encode(x, pos, p)

    c1 = p['conv1']
    ew1 = _edge_mlp(ea, c1, True)
    t1 = _sc_scatter(h0, ew1, src, dst)
    y1, cnt8 = _node_layer(t1, h0, None, c1, True)
    h1 = _bn_relu(y1, p['bn1_g'], p['bn1_b'])

    c2 = p['conv2']
    ew2 = _edge_mlp(ea, c2, False)
    t2 = _sc_scatter(h1, ew2, src, dst)
    y2 = _node_layer(t2, h1, cnt8, c2, False)
    h2 = _bn_relu(y2, p['bn2_g'], p['bn2_b'])

    c3 = p['conv3']
    ew3 = _edge_mlp(ea, c3, False)
    t3 = _sc_scatter(h2, ew3, src, dst)
    epwt3 = _pad_nodew(c3['ep_w'], 256)
    nmwt3 = _pad_nodew(c3['nm_w'], 256)
    tspec = pl.BlockSpec((NB, DP), _rowblk)
    gf = pl.pallas_call(
        _c3pool_body,
        grid=(N // NB,),
        in_specs=[tspec, tspec, tspec,
                  pl.BlockSpec((NB, 8), _rowblk),
                  pl.BlockSpec((1, 1, NB), lambda i: (i, 0, 0)),
                  pl.BlockSpec((DP, 256), _fixed),
                  pl.BlockSpec((1, 256), _fixed),
                  pl.BlockSpec((DP, 256), _fixed),
                  pl.BlockSpec((1, 256), _fixed)],
        out_specs=pl.BlockSpec((G, 256), _fixed),
        out_shape=jax.ShapeDtypeStruct((G, 256), jnp.float32),
        scratch_shapes=[pltpu.VMEM((G, 8), jnp.float32)],
    )(t3[0], t3[1], h2, cnt8, batch_row, epwt3, c3['ep_b'][None],
      nmwt3, c3['nm_b'][None])

    return gf


# speedup vs baseline: 3.7794x; 1.4674x over previous
import functools

import jax
import jax.numpy as jnp
from jax import lax
from jax.experimental import pallas as pl
from jax.experimental.pallas import tpu as pltpu
from jax.experimental.pallas import tpu_sc as plsc

N = 10000
E = 320000
G = 32
DP = 128
ONES_COL = 56

_HI = lax.Precision.HIGHEST


def _dot(a, b):
    return jnp.dot(a, b, preferred_element_type=jnp.float32, precision=_HI)


def _encode_body(x_ref, pos_ref, w1t_ref, b1_ref, w2t_ref, b2_ref, out_ref):
    pos = pos_ref[...]
    x = x_ref[...]
    r = jnp.sqrt(jnp.sum(pos * pos, axis=1, keepdims=True))
    px, py = pos[:, 0:1], pos[:, 1:2]
    inv = jnp.concatenate([r, px * px + py * py, px * py], axis=1)
    g1 = jnp.maximum(_dot(inv, w1t_ref[...]) + b1_ref[...], 0.0)
    geo = _dot(g1, w2t_ref[...]) + b2_ref[...]
    ones = jnp.ones((N, 1), jnp.float32)
    zeros = jnp.zeros((N, DP - ONES_COL - 1), jnp.float32)
    out_ref[...] = jnp.concatenate([x, geo, ones, zeros], axis=1)


def _encode(x, pos, p):
    return pl.pallas_call(
        _encode_body,
        out_shape=jax.ShapeDtypeStruct((N, DP), jnp.float32),
    )(x, pos, p['enc_w1'].T, p['enc_b1'][None], p['enc_w2'].T, p['enc_b2'][None])


EB = 8000
EG = 8
EBR = EB // EG


def _edge_mlp_body(w1bd_ref, b1_ref, mean_ref, g_ref, be_ref,
                   w2bd_ref, b2_ref, ea_ref, out_ref):
    h = _dot(ea_ref[...], w1bd_ref[...]) + b1_ref[...]
    mu = _dot(h, mean_ref[...])
    d = h - mu
    var = _dot(d * d, mean_ref[...])
    hn = d / jnp.sqrt(var + 1e-5) * g_ref[...] + be_ref[...]
    hn = jnp.maximum(hn, 0.0)
    ew = _dot(hn, w2bd_ref[...]) + b2_ref[...]
    out_ref[...] = ew.reshape(EB, DP)


def _edge_mlp(ea8, cp, set_ones):
    in_d = cp['em_w2'].shape[0]
    w1t = jnp.pad(cp['em_w1'].T, ((0, 3), (0, 0)))
    w2t = jnp.pad(cp['em_w2'].T, ((0, 0), (0, DP - in_d)))
    b2 = jnp.pad(cp['em_b2'], (0, DP - in_d))
    if set_ones:
        b2 = b2.at[ONES_COL].set(1.0)
    w1bd = jnp.kron(jnp.eye(EG, dtype=jnp.float32), w1t)
    w2bd = jnp.kron(jnp.eye(EG, dtype=jnp.float32), w2t)
    mean = jnp.kron(jnp.eye(EG, dtype=jnp.float32),
                    jnp.full((16, 16), 1.0 / 16.0, jnp.float32))
    fixed = lambda i: (0, 0)
    return pl.pallas_call(
        _edge_mlp_body,
        grid=(E // EB,),
        in_specs=[
            pl.BlockSpec((EG * 8, EG * 16), fixed),
            pl.BlockSpec((1, EG * 16), fixed),
            pl.BlockSpec((EG * 16, EG * 16), fixed),
            pl.BlockSpec((1, EG * 16), fixed),
            pl.BlockSpec((1, EG * 16), fixed),
            pl.BlockSpec((EG * 16, EG * DP), fixed),
            pl.BlockSpec((1, EG * DP), fixed),
            pl.BlockSpec((EBR, EG * 8), lambda i: (i, 0)),
        ],
        out_specs=pl.BlockSpec((EB, DP), lambda i: (i, 0)),
        out_shape=jax.ShapeDtypeStruct((E, DP), jnp.float32),
    )(w1bd, jnp.tile(cp['em_b1'], EG)[None],
      mean, jnp.tile(cp['ln_g'], EG)[None], jnp.tile(cp['ln_b'], EG)[None],
      w2bd, jnp.tile(b2, EG)[None], ea8)


CH = 40
NW = 32
EPT = E // NW
NCH = EPT // CH
ZR = 25


def _sc_scatter(h, ew, src, dst):
    mesh = plsc.VectorSubcoreMesh(core_axis_name="c", subcore_axis_name="s")

    @functools.partial(
        pl.kernel,
        out_type=jax.ShapeDtypeStruct((2, N, DP), jnp.float32),
        mesh=mesh,
        scratch_types=[
            pltpu.VMEM((EPT,), jnp.int32),
            pltpu.VMEM((CH,), jnp.int32),
            pltpu.VMEM((CH,), jnp.int32),
            pltpu.VMEM((CH, DP), jnp.float32),
            pltpu.VMEM((CH, DP), jnp.float32),
            pltpu.VMEM((CH, DP), jnp.float32),
            pltpu.VMEM((CH, DP), jnp.float32),
            pltpu.VMEM((ZR, DP), jnp.float32),
            pltpu.VMEM_SHARED((N, DP), jnp.float32),
            pltpu.SemaphoreType.DMA((2,)),
            pltpu.SemaphoreType.DMA((2,)),
            pltpu.SemaphoreType.DMA((2,)),
        ],
    )
    def k(h_hbm, ew_hbm, src_hbm, dst_hbm, t_hbm,
          src_all, dst_v0, dst_v1, ew_v0, ew_v1, rows_v0, rows_v1, z_v,
          t_sp, sem_d, sem_e, sem_g):
        c = lax.axis_index("c")
        s = lax.axis_index("s")
        dstb = (dst_v0, dst_v1)
        ewb = (ew_v0, ew_v1)
        rowb = (rows_v0, rows_v1)

        zero = jnp.zeros((16,), jnp.float32)

        def zrow(r, carry):
            for j in range(DP // 16):
                z_v[r, pl.ds(j * 16, 16)] = zero
            return carry

        lax.fori_loop(0, ZR, zrow, 0)

        def zcopy(q, carry):
            pltpu.sync_copy(z_v, t_sp.at[pl.ds(s * 625 + q * ZR, ZR)])
            return carry

        lax.fori_loop(0, 625 // ZR, zcopy, 0)

        base = c * (E // 2) + s * EPT
        pltpu.sync_copy(src_hbm.at[pl.ds(base, EPT)], src_all)
        plsc.subcore_barrier()

        def issue(i, b):
            off = base + i * CH
            pltpu.async_copy(dst_hbm.at[pl.ds(off, CH)], dstb[b], sem_d.at[b])
            pltpu.async_copy(ew_hbm.at[pl.ds(off, CH)], ewb[b], sem_e.at[b])
            pltpu.async_copy(h_hbm.at[src_all.at[pl.ds(i * CH, CH)]],
                             rowb[b], sem_g.at[b])

        def process(b):
            pltpu.make_async_copy(ew_hbm.at[pl.ds(0, CH)], ewb[b],
                                  sem_e.at[b]).wait()
            pltpu.make_async_copy(ew_hbm.at[pl.ds(0, CH)], rowb[b],
                                  sem_g.at[b]).wait()

            def prow(r, cr):
                for j in range(DP // 16):
                    sl = pl.ds(j * 16, 16)
                    rowb[b][r, sl] = rowb[b][r, sl] * ewb[b][r, sl]
                return cr

            lax.fori_loop(0, CH, prow, 0)
            pltpu.make_async_copy(dst_hbm.at[pl.ds(0, CH)], dstb[b],
                                  sem_d.at[b]).wait()
            pltpu.sync_copy(rowb[b], t_sp.at[dstb[b]], add=True)

        issue(0, 0)

        def pair(g, carry):
            issue(2 * g + 1, 1)
            process(0)

            @pl.when(2 * g + 2 < NCH)
            def _():
                issue(2 * g + 2, 0)

            process(1)
            return carry

        lax.fori_loop(0, NCH // 2, pair, 0)
        plsc.subcore_barrier()

        @pl.when(s == 0)
        def _():
            pltpu.sync_copy(t_sp, t_hbm.at[c])

    return k(h, ew, src, dst)


NB = 2000


def _aggr_node(t0, t1, cnt, h, epwt, epb, nmwt, nmb):
    t = t0 + t1
    tm = t / jnp.maximum(cnt, 1.0)
    aggr = _dot(tm, epwt) + jnp.where(cnt > 0.0, 1.0, 0.0) * epb
    return _dot(h, nmwt) + nmb + aggr


def _c1y_body(t0_ref, t1_ref, h_ref, epwt_ref, epb_ref, nmwt_ref, nmb_ref,
              y_ref, cnt_out_ref):
    t0 = t0_ref[...]
    t1 = t1_ref[...]
    cnt = t0[:, ONES_COL:ONES_COL + 1] + t1[:, ONES_COL:ONES_COL + 1]
    y_ref[...] = _aggr_node(t0, t1, cnt, h_ref[...], epwt_ref[...],
                            epb_ref[...], nmwt_ref[...], nmb_ref[...])
    cnt_out_ref[...] = jnp.broadcast_to(cnt, (NB, 8))


def _c2y_body(t0_ref, t1_ref, h_ref, cnt_ref, epwt_ref, epb_ref, nmwt_ref,
              nmb_ref, y_ref):
    cnt = cnt_ref[:, 0:1]
    y_ref[...] = _aggr_node(t0_ref[...], t1_ref[...], cnt, h_ref[...],
                            epwt_ref[...], epb_ref[...], nmwt_ref[...],
                            nmb_ref[...])


def _bn_body(y_ref, g_ref, b_ref, out_ref):
    y = y_ref[...]
    m = jnp.mean(y, axis=0, keepdims=True)
    v = jnp.mean((y - m) ** 2, axis=0, keepdims=True)
    out_ref[...] = jnp.maximum(
        (y - m) / jnp.sqrt(v + 1e-5) * g_ref[...] + b_ref[...], 0.0)


def _c3pool_body(t0_ref, t1_ref, h_ref, cnt_ref, batch_ref, epwt_ref,
                 epb_ref, nmwt_ref, nmb_ref, out_ref, cnt_acc):
    i = pl.program_id(0)
    cnt = cnt_ref[:, 0:1]
    y = _aggr_node(t0_ref[...], t1_ref[...], cnt, h_ref[...], epwt_ref[...],
                   epb_ref[...], nmwt_ref[...], nmb_ref[...])
    bid = batch_ref[0]
    gi = lax.broadcasted_iota(jnp.int32, (G, NB), 0)
    mask = (gi == bid).astype(jnp.float32)

    @pl.when(i == 0)
    def _():
        out_ref[...] = jnp.zeros_like(out_ref)
        cnt_acc[...] = jnp.zeros_like(cnt_acc)

    out_ref[...] += _dot(mask, y)
    cnt_acc[...] += jnp.broadcast_to(
        jnp.sum(mask, axis=1, keepdims=True), (G, 8))

    @pl.when(i == pl.num_programs(0) - 1)
    def _():
        out_ref[...] = out_ref[...] / jnp.maximum(cnt_acc[:, 0:1], 1.0)


def _pad_nodew(w, out_d):
    return jnp.pad(w.T, ((0, DP - w.shape[1]), (0, 0)))


def _fixed(i):
    return (0, 0)


def _rowblk(i):
    return (i, 0)


def _node_layer(t, h, cnt8, cp, first):
    out_d = cp['ep_w'].shape[0]
    epwt = _pad_nodew(cp['ep_w'], out_d)
    nmwt = _pad_nodew(cp['nm_w'], out_d)
    wspecs = [
        pl.BlockSpec((DP, out_d), _fixed),
        pl.BlockSpec((1, out_d), _fixed),
        pl.BlockSpec((DP, out_d), _fixed),
        pl.BlockSpec((1, out_d), _fixed),
    ]
    tspec = pl.BlockSpec((NB, DP), _rowblk)
    if first:
        return pl.pallas_call(
            _c1y_body,
            grid=(N // NB,),
            in_specs=[tspec, tspec, tspec] + wspecs,
            out_specs=(pl.BlockSpec((NB, out_d), _rowblk),
                       pl.BlockSpec((NB, 8), _rowblk)),
            out_shape=(jax.ShapeDtypeStruct((N, out_d), jnp.float32),
                       jax.ShapeDtypeStruct((N, 8), jnp.float32)),
        )(t[0], t[1], h, epwt, cp['ep_b'][None], nmwt, cp['nm_b'][None])
    return pl.pallas_call(
        _c2y_body,
        grid=(N // NB,),
        in_specs=[tspec, tspec, tspec, pl.BlockSpec((NB, 8), _rowblk)]
        + wspecs,
        out_specs=pl.BlockSpec((NB, out_d), _rowblk),
        out_shape=jax.ShapeDtypeStruct((N, out_d), jnp.float32),
    )(t[0], t[1], h, cnt8, epwt, cp['ep_b'][None], nmwt, cp['nm_b'][None])


def _bn_relu(y, g, b):
    return pl.pallas_call(
        _bn_body,
        out_shape=jax.ShapeDtypeStruct(y.shape, jnp.float32),
    )(y, g[None], b[None])


def kernel(x, pos, edge_attr, params, edge_index, batch):
    p = params
    src = edge_index[0]
    dst = edge_index[1]
    ea8 = jnp.pad(edge_attr, ((0, 0), (0, 3))).reshape(E // EG, EG * 8)
    batch_row = batch.astype(jnp.int32).reshape(N // NB, 1, NB)

    h0 = _encode(x, pos, p)

    c1 = p['conv1']
    ew1 = _edge_mlp(ea8, c1, True)
    t1 = _sc_scatter(h0, ew1, src, dst)
    y1, cnt8 = _node_layer(t1, h0, None, c1, True)
    h1 = _bn_relu(y1, p['bn1_g'], p['bn1_b'])

    c2 = p['conv2']
    ew2 = _edge_mlp(ea8, c2, False)
    t2 = _sc_scatter(h1, ew2, src, dst)
    y2 = _node_layer(t2, h1, cnt8, c2, False)
    h2 = _bn_relu(y2, p['bn2_g'], p['bn2_b'])

    c3 = p['conv3']
    ew3 = _edge_mlp(ea8, c3, False)
    t3 = _sc_scatter(h2, ew3, src, dst)
    epwt3 = _pad_nodew(c3['ep_w'], 256)
    nmwt3 = _pad_nodew(c3['nm_w'], 256)
    tspec = pl.BlockSpec((NB, DP), _rowblk)
    gf = pl.pallas_call(
        _c3pool_body,
        grid=(N // NB,),
        in_specs=[tspec, tspec, tspec,
                  pl.BlockSpec((NB, 8), _rowblk),
                  pl.BlockSpec((1, 1, NB), lambda i: (i, 0, 0)),
                  pl.BlockSpec((DP, 256), _fixed),
                  pl.BlockSpec((1, 256), _fixed),
                  pl.BlockSpec((DP, 256), _fixed),
                  pl.BlockSpec((1, 256), _fixed)],
        out_specs=pl.BlockSpec((G, 256), _fixed),
        out_shape=jax.ShapeDtypeStruct((G, 256), jnp.float32),
        scratch_shapes=[pltpu.VMEM((G, 8), jnp.float32)],
    )(t3[0], t3[1], h2, cnt8, batch_row, epwt3, c3['ep_b'][None],
      nmwt3, c3['nm_b'][None])

    return gf


# speedup vs baseline: 4.0333x; 1.0672x over previous
import functools

import jax
import jax.numpy as jnp
from jax import lax
from jax.experimental import pallas as pl
from jax.experimental.pallas import tpu as pltpu
from jax.experimental.pallas import tpu_sc as plsc

N = 10000
E = 320000
G = 32
DP = 128
ONES_COL = 56

_HI = lax.Precision.HIGHEST


def _dot(a, b):
    return jnp.dot(a, b, preferred_element_type=jnp.float32, precision=_HI)


def _encode_body(x_ref, pos_ref, w1t_ref, b1_ref, w2t_ref, b2_ref, out_ref):
    pos = pos_ref[...]
    x = x_ref[...]
    r = jnp.sqrt(jnp.sum(pos * pos, axis=1, keepdims=True))
    px, py = pos[:, 0:1], pos[:, 1:2]
    inv = jnp.concatenate([r, px * px + py * py, px * py], axis=1)
    g1 = jnp.maximum(_dot(inv, w1t_ref[...]) + b1_ref[...], 0.0)
    geo = _dot(g1, w2t_ref[...]) + b2_ref[...]
    ones = jnp.ones((N, 1), jnp.float32)
    zeros = jnp.zeros((N, DP - ONES_COL - 1), jnp.float32)
    out_ref[...] = jnp.concatenate([x, geo, ones, zeros], axis=1)


def _encode(x, pos, p):
    return pl.pallas_call(
        _encode_body,
        out_shape=jax.ShapeDtypeStruct((N, DP), jnp.float32),
    )(x, pos, p['enc_w1'].T, p['enc_b1'][None], p['enc_w2'].T, p['enc_b2'][None])


EB = 8000
EG = 8
EBR = EB // EG


def _edge_mlp_body(w1bd_ref, b1_ref, mean_ref, g_ref, be_ref,
                   w2bd_ref, b2_ref, ea_ref, out_ref):
    h = _dot(ea_ref[...], w1bd_ref[...]) + b1_ref[...]
    mu = _dot(h, mean_ref[...])
    d = h - mu
    var = _dot(d * d, mean_ref[...])
    hn = d / jnp.sqrt(var + 1e-5) * g_ref[...] + be_ref[...]
    hn = jnp.maximum(hn, 0.0)
    ew = _dot(hn, w2bd_ref[...]) + b2_ref[...]
    out_ref[...] = ew.reshape(EB, DP)


def _edge_mlp(ea8, cp, set_ones):
    in_d = cp['em_w2'].shape[0]
    w1t = jnp.pad(cp['em_w1'].T, ((0, 3), (0, 0)))
    w2t = jnp.pad(cp['em_w2'].T, ((0, 0), (0, DP - in_d)))
    b2 = jnp.pad(cp['em_b2'], (0, DP - in_d))
    if set_ones:
        b2 = b2.at[ONES_COL].set(1.0)
    w1bd = jnp.kron(jnp.eye(EG, dtype=jnp.float32), w1t)
    w2bd = jnp.kron(jnp.eye(EG, dtype=jnp.float32), w2t)
    mean = jnp.kron(jnp.eye(EG, dtype=jnp.float32),
                    jnp.full((16, 16), 1.0 / 16.0, jnp.float32))
    fixed = lambda i: (0, 0)
    return pl.pallas_call(
        _edge_mlp_body,
        grid=(E // EB,),
        in_specs=[
            pl.BlockSpec((EG * 8, EG * 16), fixed),
            pl.BlockSpec((1, EG * 16), fixed),
            pl.BlockSpec((EG * 16, EG * 16), fixed),
            pl.BlockSpec((1, EG * 16), fixed),
            pl.BlockSpec((1, EG * 16), fixed),
            pl.BlockSpec((EG * 16, EG * DP), fixed),
            pl.BlockSpec((1, EG * DP), fixed),
            pl.BlockSpec((EBR, EG * 8), lambda i: (i, 0)),
        ],
        out_specs=pl.BlockSpec((EB, DP), lambda i: (i, 0)),
        out_shape=jax.ShapeDtypeStruct((E, DP), jnp.float32),
    )(w1bd, jnp.tile(cp['em_b1'], EG)[None],
      mean, jnp.tile(cp['ln_g'], EG)[None], jnp.tile(cp['ln_b'], EG)[None],
      w2bd, jnp.tile(b2, EG)[None], ea8)


CH = 40
NW = 32
EPT = E // NW
NCH = EPT // CH
ZR = 25


def _sc_scatter(h, ew, src, dst):
    mesh = plsc.VectorSubcoreMesh(core_axis_name="c", subcore_axis_name="s")

    @functools.partial(
        pl.kernel,
        out_type=jax.ShapeDtypeStruct((2, N, DP), jnp.float32),
        mesh=mesh,
        scratch_types=[
            pltpu.VMEM((EPT,), jnp.int32),
            pltpu.VMEM((CH,), jnp.int32),
            pltpu.VMEM((CH,), jnp.int32),
            pltpu.VMEM((CH, DP), jnp.float32),
            pltpu.VMEM((CH, DP), jnp.float32),
            pltpu.VMEM((CH, DP), jnp.float32),
            pltpu.VMEM((CH, DP), jnp.float32),
            pltpu.VMEM((ZR, DP), jnp.float32),
            pltpu.VMEM_SHARED((N, DP), jnp.float32),
            pltpu.SemaphoreType.DMA((2,)),
            pltpu.SemaphoreType.DMA((2,)),
            pltpu.SemaphoreType.DMA((2,)),
        ],
    )
    def k(h_hbm, ew_hbm, src_hbm, dst_hbm, t_hbm,
          src_all, dst_v0, dst_v1, ew_v0, ew_v1, rows_v0, rows_v1, z_v,
          t_sp, sem_d, sem_e, sem_g):
        c = lax.axis_index("c")
        s = lax.axis_index("s")
        dstb = (dst_v0, dst_v1)
        ewb = (ew_v0, ew_v1)
        rowb = (rows_v0, rows_v1)

        zero = jnp.zeros((16,), jnp.float32)

        def zrow(r, carry):
            for j in range(DP // 16):
                z_v[r, pl.ds(j * 16, 16)] = zero
            return carry

        lax.fori_loop(0, ZR, zrow, 0)

        def zcopy(q, carry):
            pltpu.sync_copy(z_v, t_sp.at[pl.ds(s * 625 + q * ZR, ZR)])
            return carry

        lax.fori_loop(0, 625 // ZR, zcopy, 0)

        base = c * (E // 2) + s * EPT
        pltpu.sync_copy(src_hbm.at[pl.ds(base, EPT)], src_all)
        plsc.subcore_barrier()

        def issue(i, b):
            off = base + i * CH
            pltpu.async_copy(dst_hbm.at[pl.ds(off, CH)], dstb[b], sem_d.at[b])
            pltpu.async_copy(ew_hbm.at[pl.ds(off, CH)], ewb[b], sem_e.at[b])
            pltpu.async_copy(h_hbm.at[src_all.at[pl.ds(i * CH, CH)]],
                             rowb[b], sem_g.at[b])

        def process(b):
            pltpu.make_async_copy(ew_hbm.at[pl.ds(0, CH)], ewb[b],
                                  sem_e.at[b]).wait()
            pltpu.make_async_copy(ew_hbm.at[pl.ds(0, CH)], rowb[b],
                                  sem_g.at[b]).wait()

            def prow(g, cr):
                r0 = g * 8
                for k in range(8):
                    for j in range(DP // 16):
                        sl = pl.ds(j * 16, 16)
                        rowb[b][r0 + k, sl] = (rowb[b][r0 + k, sl]
                                               * ewb[b][r0 + k, sl])
                return cr

            lax.fori_loop(0, CH // 8, prow, 0)
            pltpu.make_async_copy(dst_hbm.at[pl.ds(0, CH)], dstb[b],
                                  sem_d.at[b]).wait()
            pltpu.sync_copy(rowb[b], t_sp.at[dstb[b]], add=True)

        issue(0, 0)

        def pair(g, carry):
            issue(2 * g + 1, 1)
            process(0)

            @pl.when(2 * g + 2 < NCH)
            def _():
                issue(2 * g + 2, 0)

            process(1)
            return carry

        lax.fori_loop(0, NCH // 2, pair, 0)
        plsc.subcore_barrier()

        @pl.when(s == 0)
        def _():
            pltpu.sync_copy(t_sp, t_hbm.at[c])

    return k(h, ew, src, dst)


NB = 2000


def _aggr_node(t0, t1, cnt, h, epwt, epb, nmwt, nmb):
    t = t0 + t1
    tm = t / jnp.maximum(cnt, 1.0)
    aggr = _dot(tm, epwt) + jnp.where(cnt > 0.0, 1.0, 0.0) * epb
    return _dot(h, nmwt) + nmb + aggr


def _c1y_body(t0_ref, t1_ref, h_ref, epwt_ref, epb_ref, nmwt_ref, nmb_ref,
              y_ref, cnt_out_ref):
    t0 = t0_ref[...]
    t1 = t1_ref[...]
    cnt = t0[:, ONES_COL:ONES_COL + 1] + t1[:, ONES_COL:ONES_COL + 1]
    y_ref[...] = _aggr_node(t0, t1, cnt, h_ref[...], epwt_ref[...],
                            epb_ref[...], nmwt_ref[...], nmb_ref[...])
    cnt_out_ref[...] = jnp.broadcast_to(cnt, (NB, 8))


def _c2y_body(t0_ref, t1_ref, h_ref, cnt_ref, epwt_ref, epb_ref, nmwt_ref,
              nmb_ref, y_ref):
    cnt = cnt_ref[:, 0:1]
    y_ref[...] = _aggr_node(t0_ref[...], t1_ref[...], cnt, h_ref[...],
                            epwt_ref[...], epb_ref[...], nmwt_ref[...],
                            nmb_ref[...])


def _bn_body(y_ref, g_ref, b_ref, out_ref):
    y = y_ref[...]
    m = jnp.mean(y, axis=0, keepdims=True)
    v = jnp.mean((y - m) ** 2, axis=0, keepdims=True)
    out_ref[...] = jnp.maximum(
        (y - m) / jnp.sqrt(v + 1e-5) * g_ref[...] + b_ref[...], 0.0)


def _c3pool_body(t0_ref, t1_ref, h_ref, cnt_ref, batch_ref, epwt_ref,
                 epb_ref, nmwt_ref, nmb_ref, out_ref, cnt_acc):
    i = pl.program_id(0)
    cnt = cnt_ref[:, 0:1]
    y = _aggr_node(t0_ref[...], t1_ref[...], cnt, h_ref[...], epwt_ref[...],
                   epb_ref[...], nmwt_ref[...], nmb_ref[...])
    bid = batch_ref[0]
    gi = lax.broadcasted_iota(jnp.int32, (G, NB), 0)
    mask = (gi == bid).astype(jnp.float32)

    @pl.when(i == 0)
    def _():
        out_ref[...] = jnp.zeros_like(out_ref)
        cnt_acc[...] = jnp.zeros_like(cnt_acc)

    out_ref[...] += _dot(mask, y)
    cnt_acc[...] += jnp.broadcast_to(
        jnp.sum(mask, axis=1, keepdims=True), (G, 8))

    @pl.when(i == pl.num_programs(0) - 1)
    def _():
        out_ref[...] = out_ref[...] / jnp.maximum(cnt_acc[:, 0:1], 1.0)


def _pad_nodew(w, out_d):
    return jnp.pad(w.T, ((0, DP - w.shape[1]), (0, 0)))


def _fixed(i):
    return (0, 0)


def _rowblk(i):
    return (i, 0)


def _node_layer(t, h, cnt8, cp, first):
    out_d = cp['ep_w'].shape[0]
    epwt = _pad_nodew(cp['ep_w'], out_d)
    nmwt = _pad_nodew(cp['nm_w'], out_d)
    wspecs = [
        pl.BlockSpec((DP, out_d), _fixed),
        pl.BlockSpec((1, out_d), _fixed),
        pl.BlockSpec((DP, out_d), _fixed),
        pl.BlockSpec((1, out_d), _fixed),
    ]
    tspec = pl.BlockSpec((NB, DP), _rowblk)
    if first:
        return pl.pallas_call(
            _c1y_body,
            grid=(N // NB,),
            in_specs=[tspec, tspec, tspec] + wspecs,
            out_specs=(pl.BlockSpec((NB, out_d), _rowblk),
                       pl.BlockSpec((NB, 8), _rowblk)),
            out_shape=(jax.ShapeDtypeStruct((N, out_d), jnp.float32),
                       jax.ShapeDtypeStruct((N, 8), jnp.float32)),
        )(t[0], t[1], h, epwt, cp['ep_b'][None], nmwt, cp['nm_b'][None])
    return pl.pallas_call(
        _c2y_body,
        grid=(N // NB,),
        in_specs=[tspec, tspec, tspec, pl.BlockSpec((NB, 8), _rowblk)]
        + wspecs,
        out_specs=pl.BlockSpec((NB, out_d), _rowblk),
        out_shape=jax.ShapeDtypeStruct((N, out_d), jnp.float32),
    )(t[0], t[1], h, cnt8, epwt, cp['ep_b'][None], nmwt, cp['nm_b'][None])


def _bn_relu(y, g, b):
    return pl.pallas_call(
        _bn_body,
        out_shape=jax.ShapeDtypeStruct(y.shape, jnp.float32),
    )(y, g[None], b[None])


def kernel(x, pos, edge_attr, params, edge_index, batch):
    p = params
    src = edge_index[0]
    dst = edge_index[1]
    ea8 = jnp.pad(edge_attr, ((0, 0), (0, 3))).reshape(E // EG, EG * 8)
    batch_row = batch.astype(jnp.int32).reshape(N // NB, 1, NB)

    h0 = _encode(x, pos, p)

    c1 = p['conv1']
    ew1 = _edge_mlp(ea8, c1, True)
    t1 = _sc_scatter(h0, ew1, src, dst)
    y1, cnt8 = _node_layer(t1, h0, None, c1, True)
    h1 = _bn_relu(y1, p['bn1_g'], p['bn1_b'])

    c2 = p['conv2']
    ew2 = _edge_mlp(ea8, c2, False)
    t2 = _sc_scatter(h1, ew2, src, dst)
    y2 = _node_layer(t2, h1, cnt8, c2, False)
    h2 = _bn_relu(y2, p['bn2_g'], p['bn2_b'])

    c3 = p['conv3']
    ew3 = _edge_mlp(ea8, c3, False)
    t3 = _sc_scatter(h2, ew3, src, dst)
    epwt3 = _pad_nodew(c3['ep_w'], 256)
    nmwt3 = _pad_nodew(c3['nm_w'], 256)
    tspec = pl.BlockSpec((NB, DP), _rowblk)
    gf = pl.pallas_call(
        _c3pool_body,
        grid=(N // NB,),
        in_specs=[tspec, tspec, tspec,
                  pl.BlockSpec((NB, 8), _rowblk),
                  pl.BlockSpec((1, 1, NB), lambda i: (i, 0, 0)),
                  pl.BlockSpec((DP, 256), _fixed),
                  pl.BlockSpec((1, 256), _fixed),
                  pl.BlockSpec((DP, 256), _fixed),
                  pl.BlockSpec((1, 256), _fixed)],
        out_specs=pl.BlockSpec((G, 256), _fixed),
        out_shape=jax.ShapeDtypeStruct((G, 256), jnp.float32),
        scratch_shapes=[pltpu.VMEM((G, 8), jnp.float32)],
    )(t3[0], t3[1], h2, cnt8, batch_row, epwt3, c3['ep_b'][None],
      nmwt3, c3['nm_b'][None])

    return gf
